# Initial kernel scaffold; baseline (speedup 1.0000x reference)
import jax
import jax.numpy as jnp
from jax.experimental import pallas as pl


def kernel(scores, gt_boxes, metadata):
    raise NotImplementedError("write your pallas kernel here")


# speedup vs baseline: 6.1687x; 6.1687x over previous
import functools

import numpy as np
import jax
import jax.numpy as jnp
from jax import lax
from jax.experimental import pallas as pl
from jax.experimental.pallas import tpu as pltpu

_NEG_OV = 0.3
_POS_OV = 0.7
_STRIDE = 16
_LANES = 128


def _anchor_grid(rr, cc, stride):
    w = h = float(stride)
    x_ctr = 0.5 * (w - 1.0)
    y_ctr = 0.5 * (h - 1.0)
    size = w * h
    rows = []
    for r in (0.5, 1.0, 2.0):
        ws = np.round(np.sqrt(size / r))
        hs = np.round(ws * r)
        for s in (8, 16, 32):
            wss = ws * s
            hss = hs * s
            rows.append([x_ctr - 0.5 * (wss - 1.0), y_ctr - 0.5 * (hss - 1.0),
                         x_ctr + 0.5 * (wss - 1.0), y_ctr + 0.5 * (hss - 1.0)])
    base = np.array(rows, dtype=np.float32)
    sx = np.arange(cc, dtype=np.float32) * stride
    sy = np.arange(rr, dtype=np.float32) * stride
    mx, my = np.meshgrid(sx, sy)
    shifts = np.stack([mx.ravel(), my.ravel(), mx.ravel(), my.ravel()], axis=1)
    return (base[None, :, :] + shifts[:, None, :]).reshape(-1, 4).astype(np.float32)


def _body(num_gt, sc_ref, anc_ref, labels_ref, targets_ref):
    ax1 = anc_ref[0]
    ay1 = anc_ref[1]
    ax2 = anc_ref[2]
    ay2 = anc_ref[3]
    area_a = (ax2 - ax1 + 1.0) * (ay2 - ay1 + 1.0)
    m0 = sc_ref[4, 0]
    m1 = sc_ref[4, 1]
    inside = (ax1 >= 0.0) & (ay1 >= 0.0) & (ax2 < m1) & (ay2 < m0)
    shape = ax1.shape
    neg1 = jnp.full(shape, -1.0, jnp.float32)
    zero = jnp.zeros(shape, jnp.float32)

    def step(g, c):
        best, bx1, by1, bx2, by2, anyeq = c
        gx1 = sc_ref[0, g]
        gy1 = sc_ref[1, g]
        gx2 = sc_ref[2, g]
        gy2 = sc_ref[3, g]
        iw = jnp.maximum(jnp.minimum(ax2, gx2) - jnp.maximum(ax1, gx1) + 1.0, 0.0)
        ih = jnp.maximum(jnp.minimum(ay2, gy2) - jnp.maximum(ay1, gy1) + 1.0, 0.0)
        inter = iw * ih
        area_g = (gx2 - gx1 + 1.0) * (gy2 - gy1 + 1.0)
        ov = inter / (area_a + area_g - inter)
        upd = ov > best
        best = jnp.where(upd, ov, best)
        bx1 = jnp.where(upd, gx1, bx1)
        by1 = jnp.where(upd, gy1, by1)
        bx2 = jnp.where(upd, gx2, bx2)
        by2 = jnp.where(upd, gy2, by2)
        masked = jnp.where(inside, ov, neg1)
        gmax = jnp.max(masked)
        anyeq = jnp.maximum(anyeq, jnp.where(masked == gmax, 1.0, 0.0))
        return best, bx1, by1, bx2, by2, anyeq

    init = (neg1, zero, zero, zero, zero, zero)
    best, bx1, by1, bx2, by2, anyeq = lax.fori_loop(0, num_gt, step, init)

    labels = jnp.where(best < _NEG_OV, 0.0, -1.0)
    labels = jnp.where(anyeq > 0.0, 1.0, labels)
    labels = jnp.where(best >= _POS_OV, 1.0, labels)
    labels_ref[...] = jnp.where(inside, labels, -1.0)

    ex_w = ax2 - ax1 + 1.0
    ex_h = ay2 - ay1 + 1.0
    ex_cx = ax1 + 0.5 * ex_w
    ex_cy = ay1 + 0.5 * ex_h
    gt_w = bx2 - bx1 + 1.0
    gt_h = by2 - by1 + 1.0
    gt_cx = bx1 + 0.5 * gt_w
    gt_cy = by1 + 0.5 * gt_h
    targets_ref[0] = jnp.where(inside, (gt_cx - ex_cx) / ex_w, 0.0)
    targets_ref[1] = jnp.where(inside, (gt_cy - ex_cy) / ex_h, 0.0)
    targets_ref[2] = jnp.where(inside, jnp.log(gt_w / ex_w), 0.0)
    targets_ref[3] = jnp.where(inside, jnp.log(gt_h / ex_h), 0.0)


def kernel(scores, gt_boxes, metadata):
    rr, cc = scores.shape[1], scores.shape[2]
    anchors = _anchor_grid(rr, cc, _STRIDE)
    n = anchors.shape[0]
    num_gt = gt_boxes.shape[1]
    assert num_gt <= _LANES and n % _LANES == 0
    rows = n // _LANES
    anc_planes = jnp.asarray(anchors.T.reshape(4, rows, _LANES))
    scalars = (jnp.zeros((5, _LANES), jnp.float32)
               .at[:4, :num_gt].set(gt_boxes[0].T)
               .at[4, :3].set(metadata[0]))
    labels, targets = pl.pallas_call(
        functools.partial(_body, num_gt),
        out_shape=[
            jax.ShapeDtypeStruct((rows, _LANES), jnp.float32),
            jax.ShapeDtypeStruct((4, rows, _LANES), jnp.float32),
        ],
        in_specs=[
            pl.BlockSpec(memory_space=pltpu.SMEM),
            pl.BlockSpec(memory_space=pltpu.VMEM),
        ],
    )(scalars, anc_planes)
    return labels.reshape(1, n), targets.reshape(4, n).T[None]
